# baseline (speedup 1.0000x reference)
import functools

import jax
import jax.numpy as jnp
from jax.experimental import pallas as pl
from jax.experimental.pallas import tpu as pltpu
from jax.experimental.pallas import tpu_sc as plsc

_DIM = 64
_WINDOW = 128


@functools.partial(jax.jit, static_argnums=(2,))
def _gather_rows(word_embs, idx, num_idx):
    mesh = plsc.VectorSubcoreMesh(core_axis_name="core",
                                  subcore_axis_name="subcore")

    @functools.partial(
        pl.kernel,
        out_type=jax.ShapeDtypeStruct((num_idx, _DIM), jnp.float32),
        mesh=mesh,
        compiler_params=pltpu.CompilerParams(use_tc_tiling_on_sc=False),
    )
    def gather_kernel(x_hbm, i_hbm, o_hbm):
        def body(i_vmem, o_vmem):
            pltpu.sync_copy(x_hbm.at[i_vmem.at[0]], o_vmem)

        pltpu.emit_pipeline(
            body,
            grid=(num_idx // _WINDOW,),
            in_specs=[pl.BlockSpec((1, _WINDOW), index_map=lambda i: (0, i))],
            out_specs=[pl.BlockSpec((_WINDOW, _DIM),
                                    index_map=lambda i: (i, 0))],
            core_axis_name=("core", "subcore"),
            dimension_semantics=(pltpu.PARALLEL,),
        )(i_hbm, o_hbm)

    return gather_kernel(word_embs, idx)


def kernel(obj_category, sub_category, word_embs):
    b = obj_category.shape[0]
    idx = jnp.stack(
        [obj_category.astype(jnp.int32), sub_category.astype(jnp.int32)],
        axis=1,
    ).reshape(1, 2 * b)
    rows = _gather_rows(word_embs, idx, 2 * b)
    return rows.reshape(b, 2 * _DIM)


# speedup vs baseline: 1.0043x; 1.0043x over previous
import functools

import jax
import jax.numpy as jnp
from jax import lax
from jax.experimental import pallas as pl
from jax.experimental.pallas import tpu as pltpu
from jax.experimental.pallas import tpu_sc as plsc

_DIM = 64
_WINDOW = 128
_NW = 32


@functools.partial(jax.jit, static_argnums=(2,))
def _gather_rows(word_embs, idx, num_idx):
    mesh = plsc.VectorSubcoreMesh(core_axis_name="core",
                                  subcore_axis_name="subcore")
    rows_per_w = num_idx // _NW
    nchunk = rows_per_w // _WINDOW

    @functools.partial(
        pl.kernel,
        out_type=jax.ShapeDtypeStruct((num_idx, _DIM), jnp.float32),
        mesh=mesh,
        scratch_types=[
            pltpu.VMEM((nchunk, _WINDOW), jnp.int32),
            pltpu.VMEM((rows_per_w, _DIM), jnp.float32),
            pltpu.SemaphoreType.DMA,
        ],
        compiler_params=pltpu.CompilerParams(use_tc_tiling_on_sc=False),
    )
    def gather_kernel(x_hbm, i_hbm, o_hbm, idx_v, rows_v, sem):
        wid = lax.axis_index("subcore") * 2 + lax.axis_index("core")
        pltpu.sync_copy(i_hbm.at[wid], idx_v)
        copies = []
        for j in range(nchunk):
            copies.append(pltpu.async_copy(
                x_hbm.at[idx_v.at[j]],
                rows_v.at[pl.ds(j * _WINDOW, _WINDOW)],
                sem))
        for c in copies:
            c.wait()
        pltpu.sync_copy(rows_v, o_hbm.at[pl.ds(wid * rows_per_w, rows_per_w)])

    return gather_kernel(word_embs, idx)


def kernel(obj_category, sub_category, word_embs):
    b = obj_category.shape[0]
    idx = jnp.stack(
        [obj_category.astype(jnp.int32), sub_category.astype(jnp.int32)],
        axis=1,
    ).reshape(_NW, (2 * b) // (_NW * _WINDOW), _WINDOW)
    rows = _gather_rows(word_embs, idx, 2 * b)
    return rows.reshape(b, 2 * _DIM)
